# baseline (speedup 1.0000x reference)
import numpy as np
import jax
import jax.numpy as jnp
from jax import lax
from jax.experimental import pallas as pl
from jax.experimental.pallas import tpu as pltpu
from jax.experimental.pallas import tpu_sc as plsc

FEAT_STRIDE = 16
_W_AN = np.array([8., 16., 32., 64., 128., 256., 512.])
_H_AN = np.array([8., 16., 32., 64., 128., 256., 512.])
A = 7
FH, FW = 50, 50
B, G = 4, 20
IM_H, IM_W = 800.0, 800.0
RPN_BATCHSIZE = 256
NUM_FG = int(0.5 * RPN_BATCHSIZE)
P = FH * FW
S = 2560
TOTAL = A * P
NSLOT = A * S
NW = 16
CHUNK = NSLOT // NW
NV = CHUNK // 16
NPLANE = 12
NST = 9


def _build_consts():
    base = np.stack([-(_W_AN - 1) / 2, -(_H_AN - 1) / 2,
                     (_W_AN - 1) / 2, (_H_AN - 1) / 2], axis=1)
    sx = np.arange(FW) * FEAT_STRIDE
    sy = np.arange(FH) * FEAT_STRIDE
    sxx, syy = np.meshgrid(sx, sy)
    shifts = np.stack([sxx.ravel(), syy.ravel(), sxx.ravel(), syy.ravel()], axis=1)
    all_anchors = (shifts[:, None, :] + base[None, :, :]).reshape(-1, 4).astype(np.float32)
    keep = ((all_anchors[:, 0] >= 0) & (all_anchors[:, 1] >= 0)
            & (all_anchors[:, 2] < IM_W) & (all_anchors[:, 3] < IM_H))
    inds_inside = np.nonzero(keep)[0]
    n_in = len(inds_inside)
    nsort = ((n_in + 15) // 16) * 16

    def to_ap(x):
        x = np.asarray(x)
        return x.reshape((P, A) + x.shape[1:]).swapaxes(0, 1)

    def pad(x, val):
        w = [(0, 0)] * (x.ndim - 1) + [(0, S - P)]
        return np.pad(x, w, constant_values=val)

    anch = pad(to_ap(all_anchors).transpose(2, 0, 1), 0.0)
    ax1, ay1, ax2, ay2 = anch
    ax2 = np.where(ax2 == 0.0, 15.0, ax2)
    ay2 = np.where(ay2 == 0.0, 15.0, ay2)
    aw = ax2 - ax1 + 1.0
    ah = ay2 - ay1 + 1.0
    aarea = aw * ah
    ecx = ax1 + 0.5 * aw
    ecy = ay1 + 0.5 * ah
    inside = pad(to_ap(keep.astype(np.float32)), 0.0)
    planes = np.stack([ax1, ay1, ax2, ay2, aarea, ecx, ecy,
                       1.0 / aw, 1.0 / ah, np.log(aw), np.log(ah), inside]
                      ).astype(np.float32).reshape(NPLANE, NSLOT)

    inside_flat = inside.reshape(NSLOT) > 0
    outside_idx = np.nonzero(~inside_flat)[0]

    def _draws():
        key = jax.random.key(42)
        return (jax.random.uniform(key, (B, n_in)),
                jax.random.uniform(jax.random.fold_in(key, 1), (B, n_in)))

    try:
        rand_fg, rand_bg = jax.jit(_draws, backend="cpu")()
    except Exception:
        rand_fg, rand_bg = _draws()
    rand_fg = np.asarray(rand_fg)
    rand_bg = np.asarray(rand_bg)

    def perms(rand):
        out = np.empty((B, nsort), np.int32)
        for b in range(B):
            order = np.argsort(rand[b], kind="stable")
            flat_t = np.full(TOTAL, -1, np.int64)
            flat_t[inds_inside] = np.arange(n_in)
            ap = pad(to_ap(flat_t), -1).reshape(NSLOT)
            slot_of_n = np.empty(n_in, np.int64)
            slot_of_n[ap[ap >= 0]] = np.nonzero(ap >= 0)[0]
            out[b, :n_in] = slot_of_n[order]
            out[b, n_in:] = outside_idx[:nsort - n_in]
        return out

    return planes, perms(rand_fg), perms(rand_bg), n_in, nsort


_PLANES, _PERM_FG, _PERM_BG, _N_IN, _NSORT = _build_consts()


def _sc_body(planes_hbm, gtv_hbm, permf_hbm, permb_hbm,
             labels_hbm, bt_hbm, biw_hbm, bow_hbm,
             pv, gv, ovb, st, lmax_l, lmax_all, mfull, permv, kslice, nexv,
             lmax_sh, mask_sh, kept_sh, nex_sh):
    cid = lax.axis_index("c")
    sid = lax.axis_index("s")
    off = sid * CHUNK
    f32 = jnp.float32

    for r in range(NPLANE):
        pltpu.sync_copy(planes_hbm.at[pl.ds(r * NSLOT + off, CHUNK)],
                        pv.at[pl.ds(r * CHUNK, CHUNK)])

    for bl in range(2):
        b = cid * 2 + bl
        pltpu.sync_copy(gtv_hbm.at[pl.ds(b * (G * 10 * 16), G * 10 * 16)], gv)

        def init_i(i, _):
            st[pl.ds(0 * CHUNK + i * 16, 16)] = jnp.full((16,), -1.0, f32)
            st[pl.ds(5 * CHUNK + i * 16, 16)] = jnp.full((16,), 0.0, f32)
            return 0
        lax.fori_loop(0, NV, init_i, 0)

        def g_body(g, _):
            def gld(k):
                return gv[pl.ds((g * 10 + k) * 16, 16)]
            gx1 = gld(0)
            gy1 = gld(1)
            gx2 = gld(2)
            gy2 = gld(3)
            garea = gld(4)
            gcx = gld(5)
            gcy = gld(6)
            lgw = gld(7)
            lgh = gld(8)
            valid = gld(9)

            def i_body(i, lm):
                o16 = i * 16

                def pvv(r):
                    return pv[pl.ds(r * CHUNK + o16, 16)]

                def stg(r):
                    return st[pl.ds(r * CHUNK + o16, 16)]

                def sts(r, x):
                    st[pl.ds(r * CHUNK + o16, 16)] = x

                ix = jnp.minimum(pvv(2), gx2) - jnp.maximum(pvv(0), gx1) + 1.0
                iy = jnp.minimum(pvv(3), gy2) - jnp.maximum(pvv(1), gy1) + 1.0
                inter = jnp.maximum(ix, 0.0) * jnp.maximum(iy, 0.0)
                iou = inter / (pvv(4) + garea - inter)
                ov = iou * valid
                ovb[pl.ds(g * CHUNK + o16, 16)] = ov
                mo = stg(0)
                upd = ov > mo
                sts(1, jnp.where(upd, gcx, stg(1)))
                sts(2, jnp.where(upd, gcy, stg(2)))
                sts(3, jnp.where(upd, lgw, stg(3)))
                sts(4, jnp.where(upd, lgh, stg(4)))
                sts(0, jnp.maximum(mo, ov))
                return jnp.maximum(lm, ov * pvv(11))
            lm = lax.fori_loop(0, NV, i_body, jnp.full((16,), -1.0, f32))
            lmax_l[pl.ds(g * 16, 16)] = lm
            return 0
        lax.fori_loop(0, G, g_body, 0)

        pltpu.sync_copy(lmax_l, lmax_sh.at[pl.ds(sid * (G * 16), G * 16)])
        plsc.subcore_barrier()

        pltpu.sync_copy(lmax_sh, lmax_all)

        def g2_body(g, _):
            def r_body(w, m):
                return jnp.maximum(m, lmax_all[pl.ds((w * G + g) * 16, 16)])
            m = lax.fori_loop(1, NW, r_body, lmax_all[pl.ds(g * 16, 16)])
            gtm = jnp.maximum(plsc.cummax(m),
                              lax.rev(plsc.cummax(lax.rev(m, (0,))), (0,)))
            adjv = jnp.where(gtm == 0.0, f32(1e-5), gtm)

            def i_body(i, _):
                o16 = i * 16
                sl5 = pl.ds(5 * CHUNK + o16, 16)
                k = st[sl5]
                st[sl5] = jnp.where(ovb[pl.ds(g * CHUNK + o16, 16)] == adjv, 1.0, k)
                return 0
            lax.fori_loop(0, NV, i_body, 0)
            return 0
        lax.fori_loop(0, G, g2_body, 0)

        def lab_i(i, _):
            o16 = i * 16
            ins = pv[pl.ds(11 * CHUNK + o16, 16)] > 0.0
            mo = st[pl.ds(0 * CHUNK + o16, 16)]
            lab = jnp.where(ins & (mo < 0.3), 0.0, -1.0)
            lab = jnp.where(ins & ((st[pl.ds(5 * CHUNK + o16, 16)] > 0.0)
                                   | (mo >= 0.7)), 1.0, lab)
            st[pl.ds(6 * CHUNK + o16, 16)] = lab
            st[pl.ds(7 * CHUNK + o16, 16)] = jnp.where(lab == 1.0, 1.0, 0.0)
            st[pl.ds(8 * CHUNK + o16, 16)] = jnp.where(lab == 0.0, 1.0, 0.0)
            return 0
        lax.fori_loop(0, NV, lab_i, 0)

        pltpu.sync_copy(st.at[pl.ds(7 * CHUNK, CHUNK)],
                        mask_sh.at[pl.ds(0 * NSLOT + off, CHUNK)])
        pltpu.sync_copy(st.at[pl.ds(8 * CHUNK, CHUNK)],
                        mask_sh.at[pl.ds(1 * NSLOT + off, CHUNK)])
        plsc.subcore_barrier()

        def allsum(v):
            return (plsc.cumsum(v)
                    + lax.rev(plsc.cumsum(lax.rev(v, (0,))), (0,)) - v)

        def count_full():
            def cnt(i, acc):
                return acc + mfull[pl.ds(i * 16, 16)]
            return allsum(lax.fori_loop(0, NSLOT // 16, cnt,
                                        jnp.zeros((16,), f32)))

        def select(kk):
            def sel(r, carry):
                idx = permv[pl.ds(r * 16, 16)]
                v = plsc.load_gather(mfull, [idx])
                c = plsc.cumsum(v)
                exc = c - v + carry
                kept = jnp.where((v > 0.0) & (exc < kk), 1.0, 0.0)
                plsc.store_scatter(mfull, [idx], kept)
                return carry + allsum(v)
            lax.fori_loop(0, _NSORT // 16, sel, jnp.zeros((16,), f32))

        @pl.when(sid == 0)
        def _():
            pltpu.sync_copy(permf_hbm.at[pl.ds(b * _NSORT, _NSORT)], permv)
            pltpu.sync_copy(mask_sh.at[pl.ds(0, NSLOT)], mfull)
            k_fg = jnp.minimum(jnp.full((16,), float(NUM_FG), f32), count_full())
            select(k_fg)
            pltpu.sync_copy(mfull, kept_sh.at[pl.ds(0, NSLOT)])

        @pl.when(sid == 1)
        def _():
            pltpu.sync_copy(permb_hbm.at[pl.ds(b * _NSORT, _NSORT)], permv)
            pltpu.sync_copy(mask_sh.at[pl.ds(0, NSLOT)], mfull)
            k_fg = jnp.minimum(jnp.full((16,), float(NUM_FG), f32), count_full())
            pltpu.sync_copy(mask_sh.at[pl.ds(NSLOT, NSLOT)], mfull)
            bg_tot = count_full()
            k_bg = jnp.minimum(jnp.full((16,), float(RPN_BATCHSIZE), f32) - k_fg,
                               bg_tot)
            select(k_bg)
            pltpu.sync_copy(mfull, kept_sh.at[pl.ds(NSLOT, NSLOT)])
            nex = jnp.maximum(k_fg + k_bg, 1.0)
            nexv[...] = 1.0 / nex
            pltpu.sync_copy(nexv, nex_sh)

        plsc.subcore_barrier()

        pltpu.sync_copy(kept_sh.at[pl.ds(0 * NSLOT + off, CHUNK)],
                        kslice.at[pl.ds(0, CHUNK)])
        pltpu.sync_copy(kept_sh.at[pl.ds(1 * NSLOT + off, CHUNK)],
                        kslice.at[pl.ds(CHUNK, CHUNK)])
        pltpu.sync_copy(nex_sh, nexv)
        posw = nexv[...]

        def o_body(i, _):
            o16 = i * 16

            def pvv(r):
                return pv[pl.ds(r * CHUNK + o16, 16)]

            def stg(r):
                return st[pl.ds(r * CHUNK + o16, 16)]

            fgm = stg(7) > 0.0
            bgm = stg(8) > 0.0
            lab = stg(6)
            lab = jnp.where(fgm & (kslice[pl.ds(o16, 16)] < 0.5), -1.0, lab)
            lab = jnp.where(bgm & (kslice[pl.ds(CHUNK + o16, 16)] < 0.5), -1.0, lab)
            st[pl.ds(6 * CHUNK + o16, 16)] = lab
            st[pl.ds(7 * CHUNK + o16, 16)] = jnp.where(lab == 1.0, 1.0, 0.0)
            st[pl.ds(8 * CHUNK + o16, 16)] = jnp.where(lab >= 0.0, posw, 0.0)
            ins = pvv(11)
            ovb[pl.ds(0 * CHUNK + o16, 16)] = (stg(1) - pvv(5)) * pvv(7) * ins
            ovb[pl.ds(1 * CHUNK + o16, 16)] = (stg(2) - pvv(6)) * pvv(8) * ins
            ovb[pl.ds(2 * CHUNK + o16, 16)] = (stg(3) - pvv(9)) * ins
            ovb[pl.ds(3 * CHUNK + o16, 16)] = (stg(4) - pvv(10)) * ins
            return 0
        lax.fori_loop(0, NV, o_body, 0)

        pltpu.sync_copy(st.at[pl.ds(6 * CHUNK, CHUNK)],
                        labels_hbm.at[pl.ds(b * NSLOT + off, CHUNK)])
        pltpu.sync_copy(st.at[pl.ds(7 * CHUNK, CHUNK)],
                        biw_hbm.at[pl.ds(b * NSLOT + off, CHUNK)])
        pltpu.sync_copy(st.at[pl.ds(8 * CHUNK, CHUNK)],
                        bow_hbm.at[pl.ds(b * NSLOT + off, CHUNK)])
        for k4 in range(4):
            pltpu.sync_copy(ovb.at[pl.ds(k4 * CHUNK, CHUNK)],
                            bt_hbm.at[pl.ds((b * 4 + k4) * NSLOT + off, CHUNK)])
        plsc.subcore_barrier()


def kernel(scores_w, gt_boxes, im_info, num_boxes):
    del scores_w, im_info, num_boxes
    f32 = jnp.float32

    gx1 = gt_boxes[:, :, 0]
    gy1 = gt_boxes[:, :, 1]
    gx2 = gt_boxes[:, :, 2]
    gy2 = gt_boxes[:, :, 3]
    gw = gx2 - gx1 + 1.0
    gh = gy2 - gy1 + 1.0
    gtv = jnp.stack([gx1, gy1, gx2, gy2, gw * gh,
                     gx1 + 0.5 * gw, gy1 + 0.5 * gh,
                     jnp.log(gw), jnp.log(gh),
                     ((gw > 1.0) | (gh > 1.0)).astype(f32)], axis=2)
    gtv16 = jnp.broadcast_to(gtv.reshape(B, G * 10, 1), (B, G * 10, 16)).astype(f32)

    mesh = plsc.VectorSubcoreMesh(core_axis_name="c", subcore_axis_name="s",
                                  num_cores=2, num_subcores=NW)
    run = pl.kernel(
        _sc_body,
        out_type=[
            jax.ShapeDtypeStruct((B * NSLOT,), f32),
            jax.ShapeDtypeStruct((B * 4 * NSLOT,), f32),
            jax.ShapeDtypeStruct((B * NSLOT,), f32),
            jax.ShapeDtypeStruct((B * NSLOT,), f32),
        ],
        mesh=mesh,
        compiler_params=pltpu.CompilerParams(needs_layout_passes=False),
        scratch_types=[
            pltpu.VMEM((NPLANE * CHUNK,), f32),
            pltpu.VMEM((G * 10 * 16,), f32),
            pltpu.VMEM((G * CHUNK,), f32),
            pltpu.VMEM((NST * CHUNK,), f32),
            pltpu.VMEM((G * 16,), f32),
            pltpu.VMEM((NW * G * 16,), f32),
            pltpu.VMEM((NSLOT,), f32),
            pltpu.VMEM((_NSORT,), jnp.int32),
            pltpu.VMEM((2 * CHUNK,), f32),
            pltpu.VMEM((16,), f32),
            pltpu.VMEM_SHARED((NW * G * 16,), f32),
            pltpu.VMEM_SHARED((2 * NSLOT,), f32),
            pltpu.VMEM_SHARED((2 * NSLOT,), f32),
            pltpu.VMEM_SHARED((16,), f32),
        ],
    )

    labels_p, bt_p, biw_p, bow_p = run(
        jnp.asarray(_PLANES.reshape(-1)), gtv16.reshape(-1),
        jnp.asarray(_PERM_FG.reshape(-1)), jnp.asarray(_PERM_BG.reshape(-1)))

    labels_p = labels_p.reshape(B, A, S)
    bt_p = bt_p.reshape(B, 4, A, S)
    biw_p = biw_p.reshape(B, A, S)
    bow_p = bow_p.reshape(B, A, S)
    labels_out = labels_p[:, :, :P].reshape(B, 1, A * FH, FW)
    bt_out = bt_p[:, :, :, :P].transpose(0, 2, 1, 3).reshape(B, 4 * A, FH, FW)
    biw_out = jnp.broadcast_to(biw_p[:, :, None, :P], (B, A, 4, P)).reshape(B, 4 * A, FH, FW)
    bow_out = jnp.broadcast_to(bow_p[:, :, None, :P], (B, A, 4, P)).reshape(B, 4 * A, FH, FW)
    return labels_out, bt_out, biw_out, bow_out


# speedup vs baseline: 1.0841x; 1.0841x over previous
import numpy as np
import jax
import jax.numpy as jnp
from jax import lax
from jax.experimental import pallas as pl
from jax.experimental.pallas import tpu as pltpu
from jax.experimental.pallas import tpu_sc as plsc

FEAT_STRIDE = 16
_W_AN = np.array([8., 16., 32., 64., 128., 256., 512.])
_H_AN = np.array([8., 16., 32., 64., 128., 256., 512.])
A = 7
FH, FW = 50, 50
B, G = 4, 20
IM_H, IM_W = 800.0, 800.0
RPN_BATCHSIZE = 256
NUM_FG = int(0.5 * RPN_BATCHSIZE)
P = FH * FW
S = 2560
TOTAL = A * P
NSLOT = A * S
NW = 16
CHUNK = NSLOT // NW
NV = CHUNK // 16
NPLANE = 12
NST = 9


def _build_consts():
    base = np.stack([-(_W_AN - 1) / 2, -(_H_AN - 1) / 2,
                     (_W_AN - 1) / 2, (_H_AN - 1) / 2], axis=1)
    sx = np.arange(FW) * FEAT_STRIDE
    sy = np.arange(FH) * FEAT_STRIDE
    sxx, syy = np.meshgrid(sx, sy)
    shifts = np.stack([sxx.ravel(), syy.ravel(), sxx.ravel(), syy.ravel()], axis=1)
    all_anchors = (shifts[:, None, :] + base[None, :, :]).reshape(-1, 4).astype(np.float32)
    keep = ((all_anchors[:, 0] >= 0) & (all_anchors[:, 1] >= 0)
            & (all_anchors[:, 2] < IM_W) & (all_anchors[:, 3] < IM_H))
    inds_inside = np.nonzero(keep)[0]
    n_in = len(inds_inside)
    nsort = ((n_in + 15) // 16) * 16

    def to_ap(x):
        x = np.asarray(x)
        return x.reshape((P, A) + x.shape[1:]).swapaxes(0, 1)

    def pad(x, val):
        w = [(0, 0)] * (x.ndim - 1) + [(0, S - P)]
        return np.pad(x, w, constant_values=val)

    anch = pad(to_ap(all_anchors).transpose(2, 0, 1), 0.0)
    ax1, ay1, ax2, ay2 = anch
    ax2 = np.where(ax2 == 0.0, 15.0, ax2)
    ay2 = np.where(ay2 == 0.0, 15.0, ay2)
    aw = ax2 - ax1 + 1.0
    ah = ay2 - ay1 + 1.0
    aarea = aw * ah
    ecx = ax1 + 0.5 * aw
    ecy = ay1 + 0.5 * ah
    inside = pad(to_ap(keep.astype(np.float32)), 0.0)
    planes = np.stack([ax1, ay1, ax2, ay2, aarea, ecx, ecy,
                       1.0 / aw, 1.0 / ah, np.log(aw), np.log(ah), inside]
                      ).astype(np.float32).reshape(NPLANE, NSLOT)

    inside_flat = inside.reshape(NSLOT) > 0
    outside_idx = np.nonzero(~inside_flat)[0]

    def _draws():
        key = jax.random.key(42)
        return (jax.random.uniform(key, (B, n_in)),
                jax.random.uniform(jax.random.fold_in(key, 1), (B, n_in)))

    try:
        rand_fg, rand_bg = jax.jit(_draws, backend="cpu")()
    except Exception:
        rand_fg, rand_bg = _draws()
    rand_fg = np.asarray(rand_fg)
    rand_bg = np.asarray(rand_bg)

    def perms(rand):
        out = np.empty((B, nsort), np.int32)
        for b in range(B):
            order = np.argsort(rand[b], kind="stable")
            flat_t = np.full(TOTAL, -1, np.int64)
            flat_t[inds_inside] = np.arange(n_in)
            ap = pad(to_ap(flat_t), -1).reshape(NSLOT)
            slot_of_n = np.empty(n_in, np.int64)
            slot_of_n[ap[ap >= 0]] = np.nonzero(ap >= 0)[0]
            out[b, :n_in] = slot_of_n[order]
            out[b, n_in:] = outside_idx[:nsort - n_in]
        return out

    return planes, perms(rand_fg), perms(rand_bg), n_in, nsort


_PLANES, _PERM_FG, _PERM_BG, _N_IN, _NSORT = _build_consts()


def _sc_body(planes_hbm, gtv_hbm, permf_hbm, permb_hbm,
             labels_hbm, bt_hbm, biw_hbm, bow_hbm,
             pv, gv, ovb, st, lmax_l, lmax_all, mfull, permv, kslice, nexv,
             lmax_sh, mask_sh, kept_sh, nex_sh):
    cid = lax.axis_index("c")
    sid = lax.axis_index("s")
    off = sid * CHUNK
    f32 = jnp.float32

    for r in range(NPLANE):
        pltpu.sync_copy(planes_hbm.at[pl.ds(r * NSLOT + off, CHUNK)],
                        pv.at[pl.ds(r * CHUNK, CHUNK)])

    for bl in range(2):
        b = cid * 2 + bl
        pltpu.sync_copy(gtv_hbm.at[pl.ds(b * (G * 10 * 16), G * 10 * 16)], gv)

        def init_i(i, _):
            st[pl.ds(0 * CHUNK + i * 16, 16)] = jnp.full((16,), -1.0, f32)
            st[pl.ds(5 * CHUNK + i * 16, 16)] = jnp.full((16,), 0.0, f32)
            return 0
        lax.fori_loop(0, NV, init_i, 0)

        def g_body(g, _):
            def gld(k):
                return gv[pl.ds((g * 10 + k) * 16, 16)]
            gx1 = gld(0)
            gy1 = gld(1)
            gx2 = gld(2)
            gy2 = gld(3)
            garea = gld(4)
            gcx = gld(5)
            gcy = gld(6)
            lgw = gld(7)
            lgh = gld(8)
            valid = gld(9)

            def i_body(i, lm):
                for u in range(2):
                    o16 = i * 32 + u * 16

                    def pvv(r, o16=o16):
                        return pv[pl.ds(r * CHUNK + o16, 16)]

                    def stg(r, o16=o16):
                        return st[pl.ds(r * CHUNK + o16, 16)]

                    def sts(r, x, o16=o16):
                        st[pl.ds(r * CHUNK + o16, 16)] = x

                    ix = jnp.minimum(pvv(2), gx2) - jnp.maximum(pvv(0), gx1) + 1.0
                    iy = jnp.minimum(pvv(3), gy2) - jnp.maximum(pvv(1), gy1) + 1.0
                    inter = jnp.maximum(ix, 0.0) * jnp.maximum(iy, 0.0)
                    iou = inter / (pvv(4) + garea - inter)
                    ov = iou * valid
                    ovb[pl.ds(g * CHUNK + o16, 16)] = ov
                    mo = stg(0)
                    upd = ov > mo
                    sts(1, jnp.where(upd, gcx, stg(1)))
                    sts(2, jnp.where(upd, gcy, stg(2)))
                    sts(3, jnp.where(upd, lgw, stg(3)))
                    sts(4, jnp.where(upd, lgh, stg(4)))
                    sts(0, jnp.maximum(mo, ov))
                    lm = jnp.maximum(lm, ov * pvv(11))
                return lm
            lm = lax.fori_loop(0, NV // 2, i_body, jnp.full((16,), -1.0, f32))
            lmax_l[pl.ds(g * 16, 16)] = lm
            return 0
        lax.fori_loop(0, G, g_body, 0)

        pltpu.sync_copy(lmax_l, lmax_sh.at[pl.ds(sid * (G * 16), G * 16)])
        plsc.subcore_barrier()

        pltpu.sync_copy(lmax_sh, lmax_all)

        def g2_body(g, _):
            def r_body(w, m):
                return jnp.maximum(m, lmax_all[pl.ds((w * G + g) * 16, 16)])
            m = lax.fori_loop(1, NW, r_body, lmax_all[pl.ds(g * 16, 16)])
            gtm = plsc.cummax(lax.rev(plsc.cummax(m), (0,)))
            adjv = jnp.where(gtm == 0.0, f32(1e-5), gtm)

            def i_body(i, _):
                for u in range(2):
                    o16 = i * 32 + u * 16
                    sl5 = pl.ds(5 * CHUNK + o16, 16)
                    k = st[sl5]
                    st[sl5] = jnp.where(
                        ovb[pl.ds(g * CHUNK + o16, 16)] == adjv, 1.0, k)
                return 0
            lax.fori_loop(0, NV // 2, i_body, 0)
            return 0
        lax.fori_loop(0, G, g2_body, 0)

        def lab_i(i, _):
            o16 = i * 16
            ins = pv[pl.ds(11 * CHUNK + o16, 16)] > 0.0
            mo = st[pl.ds(0 * CHUNK + o16, 16)]
            lab = jnp.where(ins & (mo < 0.3), 0.0, -1.0)
            lab = jnp.where(ins & ((st[pl.ds(5 * CHUNK + o16, 16)] > 0.0)
                                   | (mo >= 0.7)), 1.0, lab)
            st[pl.ds(6 * CHUNK + o16, 16)] = lab
            st[pl.ds(7 * CHUNK + o16, 16)] = jnp.where(lab == 1.0, 1.0, 0.0)
            st[pl.ds(8 * CHUNK + o16, 16)] = jnp.where(lab == 0.0, 1.0, 0.0)
            return 0
        lax.fori_loop(0, NV, lab_i, 0)

        pltpu.sync_copy(st.at[pl.ds(7 * CHUNK, CHUNK)],
                        mask_sh.at[pl.ds(0 * NSLOT + off, CHUNK)])
        pltpu.sync_copy(st.at[pl.ds(8 * CHUNK, CHUNK)],
                        mask_sh.at[pl.ds(1 * NSLOT + off, CHUNK)])
        plsc.subcore_barrier()

        def bcast_last(c):
            return plsc.cummax(lax.rev(c, (0,)))

        def count_full():
            def cnt(i, acc):
                o = i * 64
                return (acc + mfull[pl.ds(o, 16)] + mfull[pl.ds(o + 16, 16)]
                        + mfull[pl.ds(o + 32, 16)] + mfull[pl.ds(o + 48, 16)])
            acc = lax.fori_loop(0, NSLOT // 64, cnt, jnp.zeros((16,), f32))
            return bcast_last(plsc.cumsum(acc))

        def select(kk):
            def sel(r, carry):
                idx = permv[pl.ds(r * 16, 16)]
                v = plsc.load_gather(mfull, [idx])
                c = plsc.cumsum(v)
                exc = c - v + carry
                kept = jnp.where((v > 0.0) & (exc < kk), 1.0, 0.0)
                plsc.store_scatter(mfull, [idx], kept)
                return carry + bcast_last(c)
            lax.fori_loop(0, _NSORT // 16, sel, jnp.zeros((16,), f32))

        @pl.when(sid == 0)
        def _():
            pltpu.sync_copy(permf_hbm.at[pl.ds(b * _NSORT, _NSORT)], permv)
            pltpu.sync_copy(mask_sh.at[pl.ds(0, NSLOT)], mfull)
            k_fg = jnp.minimum(jnp.full((16,), float(NUM_FG), f32), count_full())
            select(k_fg)
            pltpu.sync_copy(mfull, kept_sh.at[pl.ds(0, NSLOT)])

        @pl.when(sid == 1)
        def _():
            pltpu.sync_copy(permb_hbm.at[pl.ds(b * _NSORT, _NSORT)], permv)
            pltpu.sync_copy(mask_sh.at[pl.ds(0, NSLOT)], mfull)
            k_fg = jnp.minimum(jnp.full((16,), float(NUM_FG), f32), count_full())
            pltpu.sync_copy(mask_sh.at[pl.ds(NSLOT, NSLOT)], mfull)
            bg_tot = count_full()
            k_bg = jnp.minimum(jnp.full((16,), float(RPN_BATCHSIZE), f32) - k_fg,
                               bg_tot)
            select(k_bg)
            pltpu.sync_copy(mfull, kept_sh.at[pl.ds(NSLOT, NSLOT)])
            nex = jnp.maximum(k_fg + k_bg, 1.0)
            nexv[...] = 1.0 / nex
            pltpu.sync_copy(nexv, nex_sh)

        plsc.subcore_barrier()

        pltpu.sync_copy(kept_sh.at[pl.ds(0 * NSLOT + off, CHUNK)],
                        kslice.at[pl.ds(0, CHUNK)])
        pltpu.sync_copy(kept_sh.at[pl.ds(1 * NSLOT + off, CHUNK)],
                        kslice.at[pl.ds(CHUNK, CHUNK)])
        pltpu.sync_copy(nex_sh, nexv)
        posw = nexv[...]

        def o_body(i, _):
            o16 = i * 16

            def pvv(r):
                return pv[pl.ds(r * CHUNK + o16, 16)]

            def stg(r):
                return st[pl.ds(r * CHUNK + o16, 16)]

            fgm = stg(7) > 0.0
            bgm = stg(8) > 0.0
            lab = stg(6)
            lab = jnp.where(fgm & (kslice[pl.ds(o16, 16)] < 0.5), -1.0, lab)
            lab = jnp.where(bgm & (kslice[pl.ds(CHUNK + o16, 16)] < 0.5), -1.0, lab)
            st[pl.ds(6 * CHUNK + o16, 16)] = lab
            st[pl.ds(7 * CHUNK + o16, 16)] = jnp.where(lab == 1.0, 1.0, 0.0)
            st[pl.ds(8 * CHUNK + o16, 16)] = jnp.where(lab >= 0.0, posw, 0.0)
            ins = pvv(11)
            ovb[pl.ds(0 * CHUNK + o16, 16)] = (stg(1) - pvv(5)) * pvv(7) * ins
            ovb[pl.ds(1 * CHUNK + o16, 16)] = (stg(2) - pvv(6)) * pvv(8) * ins
            ovb[pl.ds(2 * CHUNK + o16, 16)] = (stg(3) - pvv(9)) * ins
            ovb[pl.ds(3 * CHUNK + o16, 16)] = (stg(4) - pvv(10)) * ins
            return 0
        lax.fori_loop(0, NV, o_body, 0)

        pltpu.sync_copy(st.at[pl.ds(6 * CHUNK, CHUNK)],
                        labels_hbm.at[pl.ds(b * NSLOT + off, CHUNK)])
        pltpu.sync_copy(st.at[pl.ds(7 * CHUNK, CHUNK)],
                        biw_hbm.at[pl.ds(b * NSLOT + off, CHUNK)])
        pltpu.sync_copy(st.at[pl.ds(8 * CHUNK, CHUNK)],
                        bow_hbm.at[pl.ds(b * NSLOT + off, CHUNK)])
        for k4 in range(4):
            pltpu.sync_copy(ovb.at[pl.ds(k4 * CHUNK, CHUNK)],
                            bt_hbm.at[pl.ds((b * 4 + k4) * NSLOT + off, CHUNK)])
        plsc.subcore_barrier()


def kernel(scores_w, gt_boxes, im_info, num_boxes):
    del scores_w, im_info, num_boxes
    f32 = jnp.float32

    gx1 = gt_boxes[:, :, 0]
    gy1 = gt_boxes[:, :, 1]
    gx2 = gt_boxes[:, :, 2]
    gy2 = gt_boxes[:, :, 3]
    gw = gx2 - gx1 + 1.0
    gh = gy2 - gy1 + 1.0
    gtv = jnp.stack([gx1, gy1, gx2, gy2, gw * gh,
                     gx1 + 0.5 * gw, gy1 + 0.5 * gh,
                     jnp.log(gw), jnp.log(gh),
                     ((gw > 1.0) | (gh > 1.0)).astype(f32)], axis=2)
    gtv16 = jnp.broadcast_to(gtv.reshape(B, G * 10, 1), (B, G * 10, 16)).astype(f32)

    mesh = plsc.VectorSubcoreMesh(core_axis_name="c", subcore_axis_name="s",
                                  num_cores=2, num_subcores=NW)
    run = pl.kernel(
        _sc_body,
        out_type=[
            jax.ShapeDtypeStruct((B * NSLOT,), f32),
            jax.ShapeDtypeStruct((B * 4 * NSLOT,), f32),
            jax.ShapeDtypeStruct((B * NSLOT,), f32),
            jax.ShapeDtypeStruct((B * NSLOT,), f32),
        ],
        mesh=mesh,
        compiler_params=pltpu.CompilerParams(needs_layout_passes=False),
        scratch_types=[
            pltpu.VMEM((NPLANE * CHUNK,), f32),
            pltpu.VMEM((G * 10 * 16,), f32),
            pltpu.VMEM((G * CHUNK,), f32),
            pltpu.VMEM((NST * CHUNK,), f32),
            pltpu.VMEM((G * 16,), f32),
            pltpu.VMEM((NW * G * 16,), f32),
            pltpu.VMEM((NSLOT,), f32),
            pltpu.VMEM((_NSORT,), jnp.int32),
            pltpu.VMEM((2 * CHUNK,), f32),
            pltpu.VMEM((16,), f32),
            pltpu.VMEM_SHARED((NW * G * 16,), f32),
            pltpu.VMEM_SHARED((2 * NSLOT,), f32),
            pltpu.VMEM_SHARED((2 * NSLOT,), f32),
            pltpu.VMEM_SHARED((16,), f32),
        ],
    )

    labels_p, bt_p, biw_p, bow_p = run(
        jnp.asarray(_PLANES.reshape(-1)), gtv16.reshape(-1),
        jnp.asarray(_PERM_FG.reshape(-1)), jnp.asarray(_PERM_BG.reshape(-1)))

    labels_p = labels_p.reshape(B, A, S)
    bt_p = bt_p.reshape(B, 4, A, S)
    biw_p = biw_p.reshape(B, A, S)
    bow_p = bow_p.reshape(B, A, S)
    labels_out = labels_p[:, :, :P].reshape(B, 1, A * FH, FW)
    bt_out = bt_p[:, :, :, :P].transpose(0, 2, 1, 3).reshape(B, 4 * A, FH, FW)
    biw_out = jnp.broadcast_to(biw_p[:, :, None, :P], (B, A, 4, P)).reshape(B, 4 * A, FH, FW)
    bow_out = jnp.broadcast_to(bow_p[:, :, None, :P], (B, A, 4, P)).reshape(B, 4 * A, FH, FW)
    return labels_out, bt_out, biw_out, bow_out


# speedup vs baseline: 1.2021x; 1.1088x over previous
import numpy as np
import jax
import jax.numpy as jnp
from jax import lax
from jax.experimental import pallas as pl
from jax.experimental.pallas import tpu as pltpu
from jax.experimental.pallas import tpu_sc as plsc

FEAT_STRIDE = 16
_W_AN = np.array([8., 16., 32., 64., 128., 256., 512.])
_H_AN = np.array([8., 16., 32., 64., 128., 256., 512.])
A = 7
FH, FW = 50, 50
B, G = 4, 20
IM_H, IM_W = 800.0, 800.0
RPN_BATCHSIZE = 256
NUM_FG = int(0.5 * RPN_BATCHSIZE)
P = FH * FW
S = 2560
TOTAL = A * P
NSLOT = A * S
NW = 16
CHUNK = NSLOT // NW
NV = CHUNK // 16
NPLANE = 12
NST = 16


def _build_consts():
    base = np.stack([-(_W_AN - 1) / 2, -(_H_AN - 1) / 2,
                     (_W_AN - 1) / 2, (_H_AN - 1) / 2], axis=1)
    sx = np.arange(FW) * FEAT_STRIDE
    sy = np.arange(FH) * FEAT_STRIDE
    sxx, syy = np.meshgrid(sx, sy)
    shifts = np.stack([sxx.ravel(), syy.ravel(), sxx.ravel(), syy.ravel()], axis=1)
    all_anchors = (shifts[:, None, :] + base[None, :, :]).reshape(-1, 4).astype(np.float32)
    keep = ((all_anchors[:, 0] >= 0) & (all_anchors[:, 1] >= 0)
            & (all_anchors[:, 2] < IM_W) & (all_anchors[:, 3] < IM_H))
    inds_inside = np.nonzero(keep)[0]
    n_in = len(inds_inside)
    nsort = ((n_in + 15) // 16) * 16

    def to_ap(x):
        x = np.asarray(x)
        return x.reshape((P, A) + x.shape[1:]).swapaxes(0, 1)

    def pad(x, val):
        w = [(0, 0)] * (x.ndim - 1) + [(0, S - P)]
        return np.pad(x, w, constant_values=val)

    anch = pad(to_ap(all_anchors).transpose(2, 0, 1), 0.0)
    ax1, ay1, ax2, ay2 = anch
    ax2 = np.where(ax2 == 0.0, 15.0, ax2)
    ay2 = np.where(ay2 == 0.0, 15.0, ay2)
    aw = ax2 - ax1 + 1.0
    ah = ay2 - ay1 + 1.0
    aarea = aw * ah
    ecx = ax1 + 0.5 * aw
    ecy = ay1 + 0.5 * ah
    inside = pad(to_ap(keep.astype(np.float32)), 0.0)
    planes = np.stack([ax1, ay1, ax2, ay2, aarea, ecx, ecy,
                       1.0 / aw, 1.0 / ah, np.log(aw), np.log(ah), inside]
                      ).astype(np.float32).reshape(NPLANE, NSLOT)

    inside_flat = inside.reshape(NSLOT) > 0
    outside_idx = np.nonzero(~inside_flat)[0]

    def _draws():
        key = jax.random.key(42)
        return (jax.random.uniform(key, (B, n_in)),
                jax.random.uniform(jax.random.fold_in(key, 1), (B, n_in)))

    try:
        rand_fg, rand_bg = jax.jit(_draws, backend="cpu")()
    except Exception:
        rand_fg, rand_bg = _draws()
    rand_fg = np.asarray(rand_fg)
    rand_bg = np.asarray(rand_bg)

    def perms(rand):
        out = np.empty((B, nsort), np.int32)
        for b in range(B):
            order = np.argsort(rand[b], kind="stable")
            flat_t = np.full(TOTAL, -1, np.int64)
            flat_t[inds_inside] = np.arange(n_in)
            ap = pad(to_ap(flat_t), -1).reshape(NSLOT)
            slot_of_n = np.empty(n_in, np.int64)
            slot_of_n[ap[ap >= 0]] = np.nonzero(ap >= 0)[0]
            out[b, :n_in] = slot_of_n[order]
            out[b, n_in:] = outside_idx[:nsort - n_in]
        return out

    return planes, perms(rand_fg), perms(rand_bg), n_in, nsort


_PLANES, _PERM_FG, _PERM_BG, _N_IN, _NSORT = _build_consts()


def _sc_body(planes_hbm, gtv_hbm, permf_hbm, permb_hbm,
             labels_hbm, bt_hbm, biw_hbm, bow_hbm,
             pv, gv, ovb, st, lmax_l, lmax_all, mfull, permv, kslice, nexv,
             lmax_sh, mask_sh, kept_sh, nex_sh):
    cid = lax.axis_index("c")
    sid = lax.axis_index("s")
    off = sid * CHUNK
    f32 = jnp.float32

    for r in range(NPLANE):
        pltpu.sync_copy(planes_hbm.at[pl.ds(r * NSLOT + off, CHUNK)],
                        pv.at[pl.ds(r * CHUNK, CHUNK)])

    for bl in range(2):
        b = cid * 2 + bl
        rb = 2 + bl * 7
        pltpu.sync_copy(gtv_hbm.at[pl.ds(b * (G * 10 * 16), G * 10 * 16)], gv)

        def init_i(i, _):
            st[pl.ds(0 * CHUNK + i * 16, 16)] = jnp.full((16,), -1.0, f32)
            st[pl.ds(1 * CHUNK + i * 16, 16)] = jnp.full((16,), 0.0, f32)
            return 0
        lax.fori_loop(0, NV, init_i, 0)

        def g_body(g, _):
            def gld(k):
                return gv[pl.ds((g * 10 + k) * 16, 16)]
            gx1 = gld(0)
            gy1 = gld(1)
            gx2 = gld(2)
            gy2 = gld(3)
            garea = gld(4)
            gcx = gld(5)
            gcy = gld(6)
            lgw = gld(7)
            lgh = gld(8)
            valid = gld(9)

            def i_body(i, lm):
                for u in range(2):
                    o16 = i * 32 + u * 16

                    def pvv(r, o16=o16):
                        return pv[pl.ds(r * CHUNK + o16, 16)]

                    def stg(r, o16=o16):
                        return st[pl.ds(r * CHUNK + o16, 16)]

                    def sts(r, x, o16=o16):
                        st[pl.ds(r * CHUNK + o16, 16)] = x

                    ix = jnp.minimum(pvv(2), gx2) - jnp.maximum(pvv(0), gx1) + 1.0
                    iy = jnp.minimum(pvv(3), gy2) - jnp.maximum(pvv(1), gy1) + 1.0
                    inter = jnp.maximum(ix, 0.0) * jnp.maximum(iy, 0.0)
                    iou = inter / (pvv(4) + garea - inter)
                    ov = iou * valid
                    ovb[pl.ds(g * CHUNK + o16, 16)] = ov
                    mo = stg(0)
                    upd = ov > mo
                    sts(rb + 0, jnp.where(upd, gcx, stg(rb + 0)))
                    sts(rb + 1, jnp.where(upd, gcy, stg(rb + 1)))
                    sts(rb + 2, jnp.where(upd, lgw, stg(rb + 2)))
                    sts(rb + 3, jnp.where(upd, lgh, stg(rb + 3)))
                    sts(0, jnp.maximum(mo, ov))
                    lm = jnp.maximum(lm, ov * pvv(11))
                return lm
            lm = lax.fori_loop(0, NV // 2, i_body, jnp.full((16,), -1.0, f32))
            lmax_l[pl.ds(g * 16, 16)] = lm
            return 0
        lax.fori_loop(0, G, g_body, 0)

        pltpu.sync_copy(lmax_l, lmax_sh.at[pl.ds(sid * (G * 16), G * 16)])
        plsc.subcore_barrier()
        pltpu.sync_copy(lmax_sh, lmax_all)
        plsc.subcore_barrier()

        def g2_body(g, _):
            def r_body(w, m):
                return jnp.maximum(m, lmax_all[pl.ds((w * G + g) * 16, 16)])
            m = lax.fori_loop(1, NW, r_body, lmax_all[pl.ds(g * 16, 16)])
            gtm = plsc.cummax(lax.rev(plsc.cummax(m), (0,)))
            adjv = jnp.where(gtm == 0.0, f32(1e-5), gtm)

            def i_body(i, _):
                for u in range(2):
                    o16 = i * 32 + u * 16
                    sl5 = pl.ds(1 * CHUNK + o16, 16)
                    k = st[sl5]
                    st[sl5] = jnp.where(
                        ovb[pl.ds(g * CHUNK + o16, 16)] == adjv, 1.0, k)
                return 0
            lax.fori_loop(0, NV // 2, i_body, 0)
            return 0
        lax.fori_loop(0, G, g2_body, 0)

        def lab_i(i, _):
            o16 = i * 16
            ins = pv[pl.ds(11 * CHUNK + o16, 16)] > 0.0
            mo = st[pl.ds(0 * CHUNK + o16, 16)]
            lab = jnp.where(ins & (mo < 0.3), 0.0, -1.0)
            lab = jnp.where(ins & ((st[pl.ds(1 * CHUNK + o16, 16)] > 0.0)
                                   | (mo >= 0.7)), 1.0, lab)
            st[pl.ds((rb + 4) * CHUNK + o16, 16)] = lab
            st[pl.ds((rb + 5) * CHUNK + o16, 16)] = jnp.where(lab == 1.0, 1.0, 0.0)
            st[pl.ds((rb + 6) * CHUNK + o16, 16)] = jnp.where(lab == 0.0, 1.0, 0.0)
            return 0
        lax.fori_loop(0, NV, lab_i, 0)

        pltpu.sync_copy(st.at[pl.ds((rb + 5) * CHUNK, CHUNK)],
                        mask_sh.at[pl.ds((bl * 2 + 0) * NSLOT + off, CHUNK)])
        pltpu.sync_copy(st.at[pl.ds((rb + 6) * CHUNK, CHUNK)],
                        mask_sh.at[pl.ds((bl * 2 + 1) * NSLOT + off, CHUNK)])

    plsc.subcore_barrier()

    def bcast_last(c):
        return plsc.cummax(lax.rev(c, (0,)))

    def count_full():
        def cnt(i, acc):
            o = i * 64
            return (acc + mfull[pl.ds(o, 16)] + mfull[pl.ds(o + 16, 16)]
                    + mfull[pl.ds(o + 32, 16)] + mfull[pl.ds(o + 48, 16)])
        acc = lax.fori_loop(0, NSLOT // 64, cnt, jnp.zeros((16,), f32))
        return bcast_last(plsc.cumsum(acc))

    def select(kk):
        def sel(r, carry):
            idx = permv[pl.ds(r * 16, 16)]
            v = plsc.load_gather(mfull, [idx])
            c = plsc.cumsum(v)
            exc = c - v + carry
            kept = jnp.where((v > 0.0) & (exc < kk), 1.0, 0.0)
            plsc.store_scatter(mfull, [idx], kept)
            return carry + bcast_last(c)
        lax.fori_loop(0, _NSORT // 16, sel, jnp.zeros((16,), f32))

    for w in range(4):
        blw, rolew = w // 2, w % 2
        perm_src = permf_hbm if rolew == 0 else permb_hbm

        @pl.when(sid == w)
        def _(blw=blw, rolew=rolew, perm_src=perm_src):
            bq = cid * 2 + blw
            pltpu.sync_copy(perm_src.at[pl.ds(bq * _NSORT, _NSORT)], permv)
            pltpu.sync_copy(mask_sh.at[pl.ds((blw * 2 + 0) * NSLOT, NSLOT)], mfull)
            k_fg = jnp.minimum(jnp.full((16,), float(NUM_FG), f32), count_full())
            if rolew == 0:
                select(k_fg)
            else:
                pltpu.sync_copy(mask_sh.at[pl.ds((blw * 2 + 1) * NSLOT, NSLOT)],
                                mfull)
                bg_tot = count_full()
                k_bg = jnp.minimum(
                    jnp.full((16,), float(RPN_BATCHSIZE), f32) - k_fg, bg_tot)
                select(k_bg)
                nex = jnp.maximum(k_fg + k_bg, 1.0)
                nexv[...] = 1.0 / nex
                pltpu.sync_copy(nexv, nex_sh.at[pl.ds(blw * 16, 16)])
            pltpu.sync_copy(mfull, kept_sh.at[pl.ds(w * NSLOT, NSLOT)])

    plsc.subcore_barrier()

    for bl in range(2):
        b = cid * 2 + bl
        rb = 2 + bl * 7
        pltpu.sync_copy(kept_sh.at[pl.ds((bl * 2 + 0) * NSLOT + off, CHUNK)],
                        kslice.at[pl.ds(0, CHUNK)])
        pltpu.sync_copy(kept_sh.at[pl.ds((bl * 2 + 1) * NSLOT + off, CHUNK)],
                        kslice.at[pl.ds(CHUNK, CHUNK)])
        pltpu.sync_copy(nex_sh.at[pl.ds(bl * 16, 16)], nexv)
        posw = nexv[...]

        def o_body(i, _):
            o16 = i * 16

            def pvv(r):
                return pv[pl.ds(r * CHUNK + o16, 16)]

            def stg(r):
                return st[pl.ds(r * CHUNK + o16, 16)]

            fgm = stg(rb + 5) > 0.0
            bgm = stg(rb + 6) > 0.0
            lab = stg(rb + 4)
            lab = jnp.where(fgm & (kslice[pl.ds(o16, 16)] < 0.5), -1.0, lab)
            lab = jnp.where(bgm & (kslice[pl.ds(CHUNK + o16, 16)] < 0.5), -1.0, lab)
            st[pl.ds((rb + 4) * CHUNK + o16, 16)] = lab
            st[pl.ds((rb + 5) * CHUNK + o16, 16)] = jnp.where(lab == 1.0, 1.0, 0.0)
            st[pl.ds((rb + 6) * CHUNK + o16, 16)] = jnp.where(lab >= 0.0, posw, 0.0)
            ins = pvv(11)
            ovb[pl.ds(0 * CHUNK + o16, 16)] = (stg(rb + 0) - pvv(5)) * pvv(7) * ins
            ovb[pl.ds(1 * CHUNK + o16, 16)] = (stg(rb + 1) - pvv(6)) * pvv(8) * ins
            ovb[pl.ds(2 * CHUNK + o16, 16)] = (stg(rb + 2) - pvv(9)) * ins
            ovb[pl.ds(3 * CHUNK + o16, 16)] = (stg(rb + 3) - pvv(10)) * ins
            return 0
        lax.fori_loop(0, NV, o_body, 0)

        pltpu.sync_copy(st.at[pl.ds((rb + 4) * CHUNK, CHUNK)],
                        labels_hbm.at[pl.ds(b * NSLOT + off, CHUNK)])
        pltpu.sync_copy(st.at[pl.ds((rb + 5) * CHUNK, CHUNK)],
                        biw_hbm.at[pl.ds(b * NSLOT + off, CHUNK)])
        pltpu.sync_copy(st.at[pl.ds((rb + 6) * CHUNK, CHUNK)],
                        bow_hbm.at[pl.ds(b * NSLOT + off, CHUNK)])
        for k4 in range(4):
            pltpu.sync_copy(ovb.at[pl.ds(k4 * CHUNK, CHUNK)],
                            bt_hbm.at[pl.ds((b * 4 + k4) * NSLOT + off, CHUNK)])


def kernel(scores_w, gt_boxes, im_info, num_boxes):
    del scores_w, im_info, num_boxes
    f32 = jnp.float32

    gx1 = gt_boxes[:, :, 0]
    gy1 = gt_boxes[:, :, 1]
    gx2 = gt_boxes[:, :, 2]
    gy2 = gt_boxes[:, :, 3]
    gw = gx2 - gx1 + 1.0
    gh = gy2 - gy1 + 1.0
    gtv = jnp.stack([gx1, gy1, gx2, gy2, gw * gh,
                     gx1 + 0.5 * gw, gy1 + 0.5 * gh,
                     jnp.log(gw), jnp.log(gh),
                     ((gw > 1.0) | (gh > 1.0)).astype(f32)], axis=2)
    gtv16 = jnp.broadcast_to(gtv.reshape(B, G * 10, 1), (B, G * 10, 16)).astype(f32)

    mesh = plsc.VectorSubcoreMesh(core_axis_name="c", subcore_axis_name="s",
                                  num_cores=2, num_subcores=NW)
    run = pl.kernel(
        _sc_body,
        out_type=[
            jax.ShapeDtypeStruct((B * NSLOT,), f32),
            jax.ShapeDtypeStruct((B * 4 * NSLOT,), f32),
            jax.ShapeDtypeStruct((B * NSLOT,), f32),
            jax.ShapeDtypeStruct((B * NSLOT,), f32),
        ],
        mesh=mesh,
        compiler_params=pltpu.CompilerParams(needs_layout_passes=False),
        scratch_types=[
            pltpu.VMEM((NPLANE * CHUNK,), f32),
            pltpu.VMEM((G * 10 * 16,), f32),
            pltpu.VMEM((G * CHUNK,), f32),
            pltpu.VMEM((NST * CHUNK,), f32),
            pltpu.VMEM((G * 16,), f32),
            pltpu.VMEM((NW * G * 16,), f32),
            pltpu.VMEM((NSLOT,), f32),
            pltpu.VMEM((_NSORT,), jnp.int32),
            pltpu.VMEM((2 * CHUNK,), f32),
            pltpu.VMEM((16,), f32),
            pltpu.VMEM_SHARED((NW * G * 16,), f32),
            pltpu.VMEM_SHARED((4 * NSLOT,), f32),
            pltpu.VMEM_SHARED((4 * NSLOT,), f32),
            pltpu.VMEM_SHARED((32,), f32),
        ],
    )

    labels_p, bt_p, biw_p, bow_p = run(
        jnp.asarray(_PLANES.reshape(-1)), gtv16.reshape(-1),
        jnp.asarray(_PERM_FG.reshape(-1)), jnp.asarray(_PERM_BG.reshape(-1)))

    labels_p = labels_p.reshape(B, A, S)
    bt_p = bt_p.reshape(B, 4, A, S)
    biw_p = biw_p.reshape(B, A, S)
    bow_p = bow_p.reshape(B, A, S)
    labels_out = labels_p[:, :, :P].reshape(B, 1, A * FH, FW)
    bt_out = bt_p[:, :, :, :P].transpose(0, 2, 1, 3).reshape(B, 4 * A, FH, FW)
    biw_out = jnp.broadcast_to(biw_p[:, :, None, :P], (B, A, 4, P)).reshape(B, 4 * A, FH, FW)
    bow_out = jnp.broadcast_to(bow_p[:, :, None, :P], (B, A, 4, P)).reshape(B, 4 * A, FH, FW)
    return labels_out, bt_out, biw_out, bow_out
